# baseline (speedup 1.0000x reference)
import functools

import jax
import jax.numpy as jnp
from jax import lax
from jax.experimental import pallas as pl
from jax.experimental.pallas import tpu as pltpu

_SMOOTHING = 0.1
_CONFIDENCE = 1.0 - _SMOOTHING


def _row_pass_body(x_ref, t_ref, o_ref, o2_ref, *, num_classes):
    r, c = x_ref.shape
    ch = 128
    nch = c // ch

    mx = x_ref[:, pl.ds(0, ch)]
    for k in range(1, nch):
        mx = jnp.maximum(mx, x_ref[:, pl.ds(k * ch, ch)])
    bm = jnp.max(mx, axis=1, keepdims=True)

    sa = jnp.zeros((r, ch), jnp.float32)
    sxa = jnp.zeros((r, ch), jnp.float32)
    for k in range(nch):
        xc = x_ref[:, pl.ds(k * ch, ch)]
        sxa = sxa + xc
        sa = sa + jnp.exp(xc - bm)
    s = jnp.sum(sa, axis=1, keepdims=True)
    sx = jnp.sum(sxa, axis=1, keepdims=True)

    eps = _SMOOTHING / (num_classes - 1)
    lse = bm + jnp.log(s)
    o_ref[...] = (eps * (num_classes * lse - sx)
                  + (_CONFIDENCE - eps) * lse)

    lane_io = lax.broadcasted_iota(jnp.int32, (8, 128), 1)
    sub_io = lax.broadcasted_iota(jnp.int32, (8, 128), 0)
    acc = jnp.zeros((8, 128), jnp.float32)
    for rr in range(r):
        t_s = t_ref[rr, 0]
        toff = (t_s // 128) * 128
        g8 = (rr // 8) * 8
        blk = x_ref[pl.ds(g8, 8), pl.ds(toff, 128)]
        hit = (lane_io == t_s - toff) & (sub_io == rr - g8)
        acc = acc + jnp.where(hit, blk, 0.0)
    o2_ref[...] = jnp.sum(acc).reshape(1, 1, 1)


def _mean_body(r_ref, xt_ref, o_ref, *, num_classes, n):
    eps = _SMOOTHING / (num_classes - 1)
    o_ref[...] = (jnp.sum(r_ref[...], keepdims=True)
                  - (_CONFIDENCE - eps) * jnp.sum(xt_ref[...], keepdims=True)
                  ) * (1.0 / n)


def kernel(outputs, targets):
    n, c = outputs.shape
    r = 64 if n % 64 == 0 else n
    nb = n // r
    t2 = targets.reshape(n, 1)

    row_losses, xt_part = pl.pallas_call(
        functools.partial(_row_pass_body, num_classes=c),
        grid=(nb,),
        in_specs=[
            pl.BlockSpec((r, c), lambda i: (i, 0)),
            pl.BlockSpec((r, 1), lambda i: (i, 0), memory_space=pltpu.SMEM),
        ],
        out_specs=[
            pl.BlockSpec((r, 1), lambda i: (i, 0)),
            pl.BlockSpec((1, 1, 1), lambda i: (i, 0, 0)),
        ],
        out_shape=[
            jax.ShapeDtypeStruct((n, 1), jnp.float32),
            jax.ShapeDtypeStruct((nb, 1, 1), jnp.float32),
        ],
        compiler_params=pltpu.CompilerParams(
            dimension_semantics=("arbitrary",),
        ),
    )(outputs, t2)

    loss = pl.pallas_call(
        functools.partial(_mean_body, num_classes=c, n=n),
        out_shape=jax.ShapeDtypeStruct((1, 1), jnp.float32),
    )(row_losses, xt_part.reshape(nb, 1))
    return loss[0, 0]


# speedup vs baseline: 1.1083x; 1.1083x over previous
import functools

import jax
import jax.numpy as jnp
from jax import lax
from jax.experimental import pallas as pl
from jax.experimental.pallas import tpu as pltpu

_SMOOTHING = 0.1
_CONFIDENCE = 1.0 - _SMOOTHING


def _row_pass_body(x_ref, t_ref, o_ref, o2_ref, *, num_classes):
    r, c = x_ref.shape
    ch = 128
    nch = c // ch

    mx = x_ref[:, pl.ds(0, ch)]
    for k in range(1, nch):
        mx = jnp.maximum(mx, x_ref[:, pl.ds(k * ch, ch)])
    bm = jnp.max(mx, axis=1, keepdims=True)

    sa = jnp.zeros((r, ch), jnp.float32)
    sxa = jnp.zeros((r, ch), jnp.float32)
    for k in range(nch):
        xc = x_ref[:, pl.ds(k * ch, ch)]
        sxa = sxa + xc
        sa = sa + jnp.exp(xc - bm)
    s = jnp.sum(sa, axis=1, keepdims=True)
    sx = jnp.sum(sxa, axis=1, keepdims=True)

    eps = _SMOOTHING / (num_classes - 1)
    lse = bm + jnp.log(s)
    o_ref[...] = (eps * (num_classes * lse - sx)
                  + (_CONFIDENCE - eps) * lse)

    lane_io = lax.broadcasted_iota(jnp.int32, (8, 128), 1)
    sub_io = lax.broadcasted_iota(jnp.int32, (8, 128), 0)
    acc = jnp.zeros((8, 128), jnp.float32)
    for rr in range(r):
        t_s = t_ref[rr, 0]
        toff = (t_s // 128) * 128
        g8 = (rr // 8) * 8
        blk = x_ref[pl.ds(g8, 8), pl.ds(toff, 128)]
        hit = (lane_io == t_s - toff) & (sub_io == rr - g8)
        acc = acc + jnp.where(hit, blk, 0.0)
    o2_ref[...] = jnp.sum(acc).reshape(1, 1, 1)


def _mean_body(r_ref, xt_ref, o_ref, *, num_classes, n):
    eps = _SMOOTHING / (num_classes - 1)
    o_ref[...] = (jnp.sum(r_ref[...], keepdims=True)
                  - (_CONFIDENCE - eps) * jnp.sum(xt_ref[...], keepdims=True)
                  ) * (1.0 / n)


def kernel(outputs, targets):
    n, c = outputs.shape
    r = 128 if n % 128 == 0 else n
    nb = n // r
    t2 = targets.reshape(n, 1)

    row_losses, xt_part = pl.pallas_call(
        functools.partial(_row_pass_body, num_classes=c),
        grid=(nb,),
        in_specs=[
            pl.BlockSpec((r, c), lambda i: (i, 0)),
            pl.BlockSpec((r, 1), lambda i: (i, 0), memory_space=pltpu.SMEM),
        ],
        out_specs=[
            pl.BlockSpec((r, 1), lambda i: (i, 0)),
            pl.BlockSpec((1, 1, 1), lambda i: (i, 0, 0)),
        ],
        out_shape=[
            jax.ShapeDtypeStruct((n, 1), jnp.float32),
            jax.ShapeDtypeStruct((nb, 1, 1), jnp.float32),
        ],
        compiler_params=pltpu.CompilerParams(
            dimension_semantics=("arbitrary",),
        ),
    )(outputs, t2)

    loss = pl.pallas_call(
        functools.partial(_mean_body, num_classes=c, n=n),
        out_shape=jax.ShapeDtypeStruct((1, 1), jnp.float32),
    )(row_losses, xt_part.reshape(nb, 1))
    return loss[0, 0]


# speedup vs baseline: 1.1151x; 1.0061x over previous
import functools

import jax
import jax.numpy as jnp
from jax import lax
from jax.experimental import pallas as pl
from jax.experimental.pallas import tpu as pltpu

_SMOOTHING = 0.1
_CONFIDENCE = 1.0 - _SMOOTHING


def _row_pass_body(x_ref, t_ref, o_ref, o2_ref, *, num_classes):
    r, c = x_ref.shape
    ch = 128
    nch = c // ch

    ch1 = 256
    mx = x_ref[:, pl.ds(0, ch1)]
    for k in range(1, c // ch1):
        mx = jnp.maximum(mx, x_ref[:, pl.ds(k * ch1, ch1)])
    bm = jnp.max(mx, axis=1, keepdims=True)

    sa = jnp.zeros((r, ch), jnp.float32)
    sxa = jnp.zeros((r, ch), jnp.float32)
    for k in range(nch):
        xc = x_ref[:, pl.ds(k * ch, ch)]
        sxa = sxa + xc
        sa = sa + jnp.exp(xc - bm)
    s = jnp.sum(sa, axis=1, keepdims=True)
    sx = jnp.sum(sxa, axis=1, keepdims=True)

    eps = _SMOOTHING / (num_classes - 1)
    lse = bm + jnp.log(s)
    o_ref[...] = (eps * (num_classes * lse - sx)
                  + (_CONFIDENCE - eps) * lse)

    lane_io = lax.broadcasted_iota(jnp.int32, (8, 128), 1)
    sub_io = lax.broadcasted_iota(jnp.int32, (8, 128), 0)
    acc = jnp.zeros((8, 128), jnp.float32)
    for rr in range(r):
        t_s = t_ref[rr, 0]
        toff = (t_s // 128) * 128
        g8 = (rr // 8) * 8
        blk = x_ref[pl.ds(g8, 8), pl.ds(toff, 128)]
        hit = (lane_io == t_s - toff) & (sub_io == rr - g8)
        acc = acc + jnp.where(hit, blk, 0.0)
    o2_ref[...] = jnp.sum(acc).reshape(1, 1, 1)


def _mean_body(r_ref, xt_ref, o_ref, *, num_classes, n):
    eps = _SMOOTHING / (num_classes - 1)
    o_ref[...] = (jnp.sum(r_ref[...], keepdims=True)
                  - (_CONFIDENCE - eps) * jnp.sum(xt_ref[...], keepdims=True)
                  ) * (1.0 / n)


def kernel(outputs, targets):
    n, c = outputs.shape
    r = 128 if n % 128 == 0 else n
    nb = n // r
    t2 = targets.reshape(n, 1)

    row_losses, xt_part = pl.pallas_call(
        functools.partial(_row_pass_body, num_classes=c),
        grid=(nb,),
        in_specs=[
            pl.BlockSpec((r, c), lambda i: (i, 0)),
            pl.BlockSpec((r, 1), lambda i: (i, 0), memory_space=pltpu.SMEM),
        ],
        out_specs=[
            pl.BlockSpec((r, 1), lambda i: (i, 0)),
            pl.BlockSpec((1, 1, 1), lambda i: (i, 0, 0)),
        ],
        out_shape=[
            jax.ShapeDtypeStruct((n, 1), jnp.float32),
            jax.ShapeDtypeStruct((nb, 1, 1), jnp.float32),
        ],
        compiler_params=pltpu.CompilerParams(
            dimension_semantics=("arbitrary",),
        ),
    )(outputs, t2)

    loss = pl.pallas_call(
        functools.partial(_mean_body, num_classes=c, n=n),
        out_shape=jax.ShapeDtypeStruct((1, 1), jnp.float32),
    )(row_losses, xt_part.reshape(nb, 1))
    return loss[0, 0]
